# baseline (speedup 1.0000x reference)
import functools

import jax
import jax.numpy as jnp
from jax import lax
from jax.experimental import pallas as pl
from jax.experimental.pallas import tpu as pltpu
from jax.experimental.pallas import tpu_sc as plsc

B = 16384
F = 26
VOCAB = 1000000
D = 32
L = 16

NC = 2
NS = 16
NW = NC * NS
BPW = B // NW
BB = 64
NCHUNK = BPW // BB
NPAIR = NCHUNK // 2
ROWS = BB * F

_mesh = plsc.VectorSubcoreMesh(
    core_axis_name="c", subcore_axis_name="s", num_cores=NC, num_subcores=NS
)


@functools.partial(
    pl.kernel,
    out_type=jax.ShapeDtypeStruct((B,), jnp.float32),
    mesh=_mesh,
    compiler_params=pltpu.CompilerParams(
        needs_layout_passes=False, use_tc_tiling_on_sc=False
    ),
    scratch_types=[
        pltpu.VMEM((BB, F), jnp.int32),
        pltpu.VMEM((BB, F), jnp.int32),
        pltpu.VMEM((ROWS, D), jnp.float32),
        pltpu.VMEM((ROWS, D), jnp.float32),
        pltpu.VMEM((BB * 32,), jnp.float32),
        pltpu.VMEM((BB * 32,), jnp.float32),
        pltpu.VMEM((BB,), jnp.float32),
        pltpu.SemaphoreType.DMA,
        pltpu.SemaphoreType.DMA,
        pltpu.SemaphoreType.DMA,
        pltpu.SemaphoreType.DMA,
        pltpu.SemaphoreType.DMA,
        pltpu.SemaphoreType.DMA,
        pltpu.SemaphoreType.DMA,
    ],
)
def _fm_sc(idx_hbm, bias_hbm, vect_hbm, out_hbm, idx_v0, idx_v1, rows_v0,
           rows_v1, bias_v0, bias_v1, out_v, sem_i0, sem_i1, sem_v0, sem_v1,
           sem_b0, sem_b1, sem_o):
    wid = lax.axis_index("s") * NC + lax.axis_index("c")
    lanes = lax.iota(jnp.int32, L)
    idx_v = (idx_v0, idx_v1)
    rows_v = (rows_v0, rows_v1)
    bias_v = (bias_v0, bias_v1)
    sem_i = (sem_i0, sem_i1)
    sem_v = (sem_v0, sem_v1)
    sem_b = (sem_b0, sem_b1)

    def stage_fire(c, b):
        r0 = wid * BPW + c * BB
        pltpu.async_copy(
            idx_hbm.at[pl.ds(r0, BB), :], idx_v[b], sem_i[b]
        )
        pltpu.make_async_copy(
            idx_hbm.at[pl.ds(r0, BB), :], idx_v[b], sem_i[b]
        ).wait()
        for r in range(BB):
            pltpu.async_copy(
                vect_hbm.at[idx_v[b].at[r]],
                rows_v[b].at[pl.ds(r * F, F)], sem_v[b]
            )
            pltpu.async_copy(
                bias_hbm.at[idx_v[b].at[r]],
                bias_v[b].at[pl.ds(r * 32, F)], sem_b[b]
            )

    def wait_gathers(b):
        for r in range(BB):
            pltpu.make_async_copy(
                vect_hbm.at[idx_v[b].at[r]],
                rows_v[b].at[pl.ds(r * F, F)], sem_v[b]
            ).wait()
            pltpu.make_async_copy(
                bias_hbm.at[idx_v[b].at[r]],
                bias_v[b].at[pl.ds(r * 32, F)], sem_b[b]
            ).wait()

    def compute(c, b):
        rv = rows_v[b]
        bv = bias_v[b]

        def g_body(g, carry2):
            acc = jnp.zeros((L,), jnp.float32)
            for k in range(L):
                row = (g * L + k) * F
                s0 = jnp.zeros((L,), jnp.float32)
                s1 = jnp.zeros((L,), jnp.float32)
                q0 = jnp.zeros((L,), jnp.float32)
                q1 = jnp.zeros((L,), jnp.float32)
                for f in range(F):
                    v0 = rv[row + f, pl.ds(0, L)]
                    v1 = rv[row + f, pl.ds(L, L)]
                    s0 = s0 + v0
                    s1 = s1 + v1
                    q0 = q0 + v0 * v0
                    q1 = q1 + v1 * v1
                t = 0.5 * ((s0 * s0 - q0) + (s1 * s1 - q1))
                brow = (g * L + k) * 32
                b0 = bv[pl.ds(brow, L)]
                b1 = bv[pl.ds(brow + F - L, L)]
                b1 = jnp.where(lanes >= 2 * L - F, b1, 0.0)
                acc = jnp.where(lanes == k, jnp.sum(t + b0 + b1), acc)
            out_v[pl.ds(g * L, L)] = acc
            return carry2

        lax.fori_loop(0, BB // L, g_body, 0)
        pltpu.async_copy(
            out_v, out_hbm.at[pl.ds(wid * BPW + c * BB, BB)], sem_o
        ).wait()

    stage_fire(0, 0)

    def pair_body(c2, carry):
        c = 2 * c2
        stage_fire(c + 1, 1)
        wait_gathers(0)
        compute(c, 0)
        pl.when(c2 + 1 < NPAIR)(lambda: stage_fire(c + 2, 0))
        wait_gathers(1)
        compute(c + 1, 1)
        return carry

    lax.fori_loop(0, NPAIR, pair_body, 0)


def kernel(input, bias_w, vect_w):
    one = (input[0, 0] * 0 + 1).astype(jnp.float32)
    return _fm_sc(input.astype(jnp.int32), bias_w[:, 0] * one, vect_w)


# speedup vs baseline: 1.0263x; 1.0263x over previous
import functools

import jax
import jax.numpy as jnp
from jax import lax
from jax.experimental import pallas as pl
from jax.experimental.pallas import tpu as pltpu
from jax.experimental.pallas import tpu_sc as plsc

B = 16384
F = 26
VOCAB = 1000000
D = 32
L = 16

NC = 2
NS = 16
NW = NC * NS
BPW = B // NW
BB = 64
NCHUNK = BPW // BB
NPAIR = NCHUNK // 2
ROWS = BB * F
GSZ = 128
NG = ROWS // GSZ

_mesh = plsc.VectorSubcoreMesh(
    core_axis_name="c", subcore_axis_name="s", num_cores=NC, num_subcores=NS
)


@functools.partial(
    pl.kernel,
    out_type=jax.ShapeDtypeStruct((B,), jnp.float32),
    mesh=_mesh,
    compiler_params=pltpu.CompilerParams(
        needs_layout_passes=False, use_tc_tiling_on_sc=False
    ),
    scratch_types=[
        pltpu.VMEM((NG, GSZ), jnp.int32),
        pltpu.VMEM((NG, GSZ), jnp.int32),
        pltpu.VMEM((ROWS, D), jnp.float32),
        pltpu.VMEM((ROWS, D), jnp.float32),
        pltpu.VMEM((ROWS,), jnp.float32),
        pltpu.VMEM((ROWS,), jnp.float32),
        pltpu.VMEM((BB,), jnp.float32),
        pltpu.SemaphoreType.DMA,
        pltpu.SemaphoreType.DMA,
        pltpu.SemaphoreType.DMA,
        pltpu.SemaphoreType.DMA,
        pltpu.SemaphoreType.DMA,
        pltpu.SemaphoreType.DMA,
        pltpu.SemaphoreType.DMA,
    ],
)
def _fm_sc(idx_hbm, bias_hbm, vect_hbm, out_hbm, idx_v0, idx_v1, rows_v0,
           rows_v1, bias_v0, bias_v1, out_v, sem_i0, sem_i1, sem_v0, sem_v1,
           sem_b0, sem_b1, sem_o):
    wid = lax.axis_index("s") * NC + lax.axis_index("c")
    lanes = lax.iota(jnp.int32, L)
    idx_v = (idx_v0, idx_v1)
    rows_v = (rows_v0, rows_v1)
    bias_v = (bias_v0, bias_v1)
    sem_i = (sem_i0, sem_i1)
    sem_v = (sem_v0, sem_v1)
    sem_b = (sem_b0, sem_b1)

    def stage_fire(c, b):
        off0 = (wid * BPW + c * BB) * F
        for j in range(NG):
            pltpu.async_copy(
                idx_hbm.at[pl.ds(off0 + j * GSZ, GSZ)], idx_v[b].at[j],
                sem_i[b]
            )
        for j in range(NG):
            pltpu.make_async_copy(
                idx_hbm.at[pl.ds(off0 + j * GSZ, GSZ)], idx_v[b].at[j],
                sem_i[b]
            ).wait()
        for j in range(NG):
            pltpu.async_copy(
                vect_hbm.at[idx_v[b].at[j]],
                rows_v[b].at[pl.ds(j * GSZ, GSZ)], sem_v[b]
            )
            pltpu.async_copy(
                bias_hbm.at[idx_v[b].at[j]],
                bias_v[b].at[pl.ds(j * GSZ, GSZ)], sem_b[b]
            )

    def wait_gathers(b):
        for j in range(NG):
            pltpu.make_async_copy(
                vect_hbm.at[idx_v[b].at[j]],
                rows_v[b].at[pl.ds(j * GSZ, GSZ)], sem_v[b]
            ).wait()
            pltpu.make_async_copy(
                bias_hbm.at[idx_v[b].at[j]],
                bias_v[b].at[pl.ds(j * GSZ, GSZ)], sem_b[b]
            ).wait()

    def compute(c, b):
        rv = rows_v[b]
        bv = bias_v[b]

        def g_body(g, carry2):
            acc = jnp.zeros((L,), jnp.float32)
            for k in range(L):
                row = (g * L + k) * F
                s0 = jnp.zeros((L,), jnp.float32)
                s1 = jnp.zeros((L,), jnp.float32)
                q0 = jnp.zeros((L,), jnp.float32)
                q1 = jnp.zeros((L,), jnp.float32)
                for f in range(F):
                    v0 = rv[row + f, pl.ds(0, L)]
                    v1 = rv[row + f, pl.ds(L, L)]
                    s0 = s0 + v0
                    s1 = s1 + v1
                    q0 = q0 + v0 * v0
                    q1 = q1 + v1 * v1
                t = 0.5 * ((s0 * s0 - q0) + (s1 * s1 - q1))
                b0 = bv[pl.ds(row, L)]
                b1 = bv[pl.ds(row + F - L, L)]
                b1 = jnp.where(lanes >= 2 * L - F, b1, 0.0)
                acc = jnp.where(lanes == k, jnp.sum(t + b0 + b1), acc)
            out_v[pl.ds(g * L, L)] = acc
            return carry2

        lax.fori_loop(0, BB // L, g_body, 0)
        pltpu.async_copy(
            out_v, out_hbm.at[pl.ds(wid * BPW + c * BB, BB)], sem_o
        ).wait()

    stage_fire(0, 0)

    def pair_body(c2, carry):
        c = 2 * c2
        stage_fire(c + 1, 1)
        wait_gathers(0)
        compute(c, 0)
        pl.when(c2 + 1 < NPAIR)(lambda: stage_fire(c + 2, 0))
        wait_gathers(1)
        compute(c + 1, 1)
        return carry

    lax.fori_loop(0, NPAIR, pair_body, 0)


def kernel(input, bias_w, vect_w):
    idx = input.astype(jnp.int32).reshape(B * F)
    bias = bias_w.reshape(VOCAB)
    return _fm_sc(idx, bias, vect_w)
